# baseline (speedup 1.0000x reference)
import functools

import jax
import jax.numpy as jnp
from jax import lax
from jax.experimental import pallas as pl
from jax.experimental.pallas import tpu as pltpu
from jax.experimental.pallas import tpu_sc as plsc

N_NODES = 10000
N_EDGES = 320000
CH = 128

NC = 2
NS = 16
NL = 16
NW = NC * NS
EPW = N_EDGES // NW
CHUNK = 128
ROWS_PER_TILE = 624
TAIL_ROWS = N_NODES - NS * ROWS_PER_TILE

_PARTS = (
    dict(gn=42, sn=21, tail=0),
    dict(gn=42, sn=21, tail=0),
    dict(gn=42, sn=21, tail=0),
    dict(gn=30, sn=15, tail=16),
)
_EDGE_BLK = 2048

_sc_mesh = plsc.VectorSubcoreMesh(core_axis_name="c", subcore_axis_name="s")


GC = 64
_RING = 6
_LOOK = 4


def _make_gather(nfull, tail, start):
    epw = nfull * GC + tail
    npart = NW * epw

    @functools.partial(
        pl.kernel,
        out_type=jax.ShapeDtypeStruct((npart, CH), jnp.float32),
        mesh=_sc_mesh,
        scratch_types=[
            pltpu.VMEM((_RING, GC), jnp.int32),
            pltpu.VMEM((_RING, GC), jnp.int32),
            pltpu.VMEM((_RING, GC, CH), jnp.float32),
            pltpu.VMEM((_RING, GC, CH), jnp.float32),
        ] + [pltpu.SemaphoreType.DMA] * (2 * _RING),
    )
    def gather_part(xa_hbm, xb_hbm, row_hbm, col_hbm, out_hbm,
                    idx_a, idx_b, buf_a, buf_b, *sems):
        sems_g = sems[:_RING]
        sems_w = sems[_RING:]
        wid = lax.axis_index("s") * NC + lax.axis_index("c")
        gbase = start + wid * epw
        obase = wid * epw

        def load_idx(j, b):
            off = gbase + j * GC
            pltpu.sync_copy(row_hbm.at[pl.ds(off, GC)], idx_a.at[b])
            pltpu.sync_copy(col_hbm.at[pl.ds(off, GC)], idx_b.at[b])

        def issue_gather(b):
            pltpu.async_copy(xa_hbm.at[idx_a.at[b]], buf_a.at[b], sems_g[b])
            pltpu.async_copy(xb_hbm.at[idx_b.at[b]], buf_b.at[b], sems_g[b])

        def wait_gather(b):
            pltpu.make_async_copy(xa_hbm.at[idx_a.at[b]], buf_a.at[b],
                                  sems_g[b]).wait()
            pltpu.make_async_copy(xb_hbm.at[idx_b.at[b]], buf_b.at[b],
                                  sems_g[b]).wait()

        def issue_write(j, b):
            off = obase + j * GC
            pltpu.async_copy(buf_a.at[b], out_hbm.at[pl.ds(off, GC)], sems_w[b])

        def wait_write(j, b):
            off = obase + j * GC
            pltpu.make_async_copy(buf_a.at[b], out_hbm.at[pl.ds(off, GC)],
                                  sems_w[b]).wait()

        def add_rows(b, n):
            def add_row(r, c2):
                for g in range(CH // NL):
                    sl = pl.ds(g * NL, NL)
                    buf_a[b, r, sl] = buf_a[b, r, sl] + buf_b[b, r, sl]
                return c2
            lax.fori_loop(0, n, add_row, 0, unroll=4)

        for b in range(_LOOK):
            load_idx(b, b)
            issue_gather(b)

        def group_body(g, carry):
            for b in range(_RING):
                j = g * _RING + b
                bn = (b + _LOOK) % _RING

                @pl.when(j + _LOOK < nfull)
                def _refill():
                    @pl.when(j >= _RING - _LOOK)
                    def _drain_prev_write():
                        wait_write(j - (_RING - _LOOK), bn)
                    load_idx(j + _LOOK, bn)
                    issue_gather(bn)

                wait_gather(b)
                add_rows(b, GC)
                issue_write(j, b)
            return carry

        lax.fori_loop(0, nfull // _RING, group_body, 0)
        for b in range(_RING):
            wait_write(nfull - _RING + b, b)

        if tail:
            toff_g = gbase + nfull * GC
            toff_o = obase + nfull * GC
            tsl = pl.ds(0, tail)
            pltpu.sync_copy(row_hbm.at[pl.ds(toff_g, tail)], idx_a.at[0, tsl])
            pltpu.sync_copy(col_hbm.at[pl.ds(toff_g, tail)], idx_b.at[0, tsl])
            pltpu.async_copy(xa_hbm.at[idx_a.at[0, tsl]], buf_a.at[0, tsl],
                             sems_g[0]).wait()
            pltpu.async_copy(xb_hbm.at[idx_b.at[0, tsl]], buf_b.at[0, tsl],
                             sems_g[1]).wait()
            add_rows(0, tail)
            pltpu.sync_copy(buf_a.at[0, tsl], out_hbm.at[pl.ds(toff_o, tail)])

    return gather_part


def _make_scatter(nfull, tail, start):
    epw = nfull * CHUNK + tail
    npart = NW * epw
    neven = (nfull // 2) * 2

    @functools.partial(
        pl.kernel,
        out_type=jax.ShapeDtypeStruct((NC, N_NODES, CH), jnp.float32),
        mesh=_sc_mesh,
        scratch_types=[
            pltpu.VMEM((2, CHUNK), jnp.int32),
            pltpu.VMEM((16,), jnp.int32),
            pltpu.VMEM((2, CHUNK, CH), jnp.float32),
            pltpu.VMEM_SHARED((N_NODES, CH), jnp.float32),
            pltpu.SemaphoreType.DMA,
            pltpu.SemaphoreType.DMA,
        ],
    )
    def scatter_part(gated_hbm, col_hbm, out_hbm, idx_v, idx_t, buf, acc,
                     ss0, ss1):
        sems = (ss0, ss1)
        cid = lax.axis_index("c")
        sid = lax.axis_index("s")
        wid = sid * NC + cid

        def zrow(r, c2):
            for g in range(CH // NL):
                buf[0, r, pl.ds(g * NL, NL)] = jnp.zeros((NL,), jnp.float32)
            return c2

        lax.fori_loop(0, CHUNK, zrow, 0)
        for k in range(ROWS_PER_TILE // CHUNK):
            pltpu.sync_copy(buf.at[0],
                            acc.at[pl.ds(sid * ROWS_PER_TILE + k * CHUNK,
                                         CHUNK)])
        _zrem = ROWS_PER_TILE % CHUNK
        pltpu.sync_copy(buf.at[0, pl.ds(0, _zrem)],
                        acc.at[pl.ds(sid * ROWS_PER_TILE + ROWS_PER_TILE
                                     - _zrem, _zrem)])

        @pl.when(sid == 0)
        def _zero_tail():
            pltpu.sync_copy(buf.at[0, pl.ds(0, TAIL_ROWS)],
                            acc.at[pl.ds(NS * ROWS_PER_TILE, TAIL_ROWS)])

        plsc.subcore_barrier()

        gbase = start + wid * epw
        pbase = wid * epw

        def wait_scatter(b):
            pltpu.make_async_copy(buf.at[b], acc.at[idx_v.at[b]],
                                  sems[b]).wait()

        def chunk_body(j, carry):
            for b in range(2):
                jj = j * 2 + b

                @pl.when(jj >= 2)
                def _drain():
                    wait_scatter(b)

                pltpu.sync_copy(col_hbm.at[pl.ds(gbase + jj * CHUNK, CHUNK)],
                                idx_v.at[b])
                pltpu.sync_copy(gated_hbm.at[pl.ds(pbase + jj * CHUNK, CHUNK)],
                                buf.at[b])
                pltpu.async_copy(buf.at[b], acc.at[idx_v.at[b]], sems[b],
                                 add=True)
            return carry

        lax.fori_loop(0, neven // 2, chunk_body, 0)
        for b in range(2):
            wait_scatter(b)

        if nfull % 2:
            jj = nfull - 1
            pltpu.sync_copy(col_hbm.at[pl.ds(gbase + jj * CHUNK, CHUNK)],
                            idx_v.at[0])
            pltpu.sync_copy(gated_hbm.at[pl.ds(pbase + jj * CHUNK, CHUNK)],
                            buf.at[0])
            pltpu.sync_copy(buf.at[0], acc.at[idx_v.at[0]], add=True)

        if tail:
            tsl = pl.ds(0, tail)
            pltpu.sync_copy(col_hbm.at[pl.ds(gbase + nfull * CHUNK, tail)],
                            idx_t)
            pltpu.sync_copy(gated_hbm.at[pl.ds(pbase + nfull * CHUNK, tail)],
                            buf.at[0, tsl])
            pltpu.sync_copy(buf.at[0, tsl], acc.at[idx_t], add=True)

        plsc.subcore_barrier()
        pltpu.sync_copy(acc.at[pl.ds(sid * ROWS_PER_TILE, ROWS_PER_TILE)],
                        out_hbm.at[cid, pl.ds(sid * ROWS_PER_TILE,
                                              ROWS_PER_TILE)])

        @pl.when(sid == 0)
        def _copy_tail():
            pltpu.sync_copy(acc.at[pl.ds(NS * ROWS_PER_TILE, TAIL_ROWS)],
                            out_hbm.at[cid, pl.ds(NS * ROWS_PER_TILE,
                                                  TAIL_ROWS)])

    return scatter_part


_NODE_BLK = 2000


def _tc_node_body(x_ref, w1s_ref, w1t_ref, xa_ref, xb_ref):
    xv = x_ref[...]
    xa_ref[...] = jnp.dot(xv, w1s_ref[...], preferred_element_type=jnp.float32)
    xb_ref[...] = jnp.dot(xv, w1t_ref[...], preferred_element_type=jnp.float32)


def _tc_node(x, w1s, w1t):
    n_blk = N_NODES // _NODE_BLK
    return pl.pallas_call(
        _tc_node_body,
        grid=(n_blk,),
        in_specs=[
            pl.BlockSpec((_NODE_BLK, CH), lambda i: (i, 0)),
            pl.BlockSpec((CH, CH), lambda i: (0, 0)),
            pl.BlockSpec((CH, CH), lambda i: (0, 0)),
        ],
        out_specs=[
            pl.BlockSpec((_NODE_BLK, CH), lambda i: (i, 0)),
            pl.BlockSpec((_NODE_BLK, CH), lambda i: (i, 0)),
        ],
        out_shape=[
            jax.ShapeDtypeStruct((N_NODES, CH), jnp.float32),
            jax.ShapeDtypeStruct((N_NODES, CH), jnp.float32),
        ],
        compiler_params=pltpu.CompilerParams(
            dimension_semantics=("parallel",)),
    )(x, w1s, w1t)


def _tc_edge_body(ea_ref, g_ref, conf_ref, w1e_ref, b1_ref, w2_ref,
                  b2_ref, wa_ref, s_ref, out_ref):
    pre = (jnp.dot(ea_ref[...], w1e_ref[...], preferred_element_type=jnp.float32)
           + g_ref[...] + b1_ref[...])
    h = jnp.maximum(pre, 0.0)
    msg = jnp.dot(h, w2_ref[...], preferred_element_type=jnp.float32) + b2_ref[...]
    logit = (jnp.dot(msg, wa_ref[...], preferred_element_type=jnp.float32)
             + conf_ref[...] * s_ref[0, 0] + s_ref[0, 1])
    out_ref[...] = msg * jax.nn.sigmoid(logit)


def _tc_edge(npart, start_blk):
    n_blk = (npart + _EDGE_BLK - 1) // _EDGE_BLK

    def call(edge_attr, g, conf, w1e, b1, w2, b2, wa1, scal):
        return pl.pallas_call(
            _tc_edge_body,
            grid=(n_blk,),
            in_specs=[
                pl.BlockSpec((_EDGE_BLK, CH), lambda i: (i + start_blk, 0)),
                pl.BlockSpec((_EDGE_BLK, CH), lambda i: (i, 0)),
                pl.BlockSpec((_EDGE_BLK, 1), lambda i: (i + start_blk, 0)),
                pl.BlockSpec((CH, CH), lambda i: (0, 0)),
                pl.BlockSpec((1, CH), lambda i: (0, 0)),
                pl.BlockSpec((CH, CH), lambda i: (0, 0)),
                pl.BlockSpec((1, CH), lambda i: (0, 0)),
                pl.BlockSpec((CH, 1), lambda i: (0, 0)),
                pl.BlockSpec(memory_space=pltpu.SMEM),
            ],
            out_specs=pl.BlockSpec((_EDGE_BLK, CH), lambda i: (i, 0)),
            out_shape=jax.ShapeDtypeStruct((npart, CH), jnp.float32),
            compiler_params=pltpu.CompilerParams(
                dimension_semantics=("parallel",)),
        )(edge_attr, g, conf, w1e, b1, w2, b2, wa1, scal)

    return call


def _tc_add_body(p0_ref, p1_ref, p2_ref, p3_ref, out_ref):
    out_ref[...] = ((p0_ref[0] + p0_ref[1]) + (p1_ref[0] + p1_ref[1])
                    + ((p2_ref[0] + p2_ref[1]) + (p3_ref[0] + p3_ref[1])))


def _tc_add(partials):
    n_blk = N_NODES // _NODE_BLK
    spec = pl.BlockSpec((NC, _NODE_BLK, CH), lambda i: (0, i, 0))
    return pl.pallas_call(
        _tc_add_body,
        grid=(n_blk,),
        in_specs=[spec, spec, spec, spec],
        out_specs=pl.BlockSpec((_NODE_BLK, CH), lambda i: (i, 0)),
        out_shape=jax.ShapeDtypeStruct((N_NODES, CH), jnp.float32),
        compiler_params=pltpu.CompilerParams(
            dimension_semantics=("parallel",)),
    )(*partials)


def _build_parts():
    parts = []
    start = 0
    for p in _PARTS:
        epw = p["sn"] * CHUNK + p["tail"]
        npart = NW * epw
        parts.append(dict(
            start=start,
            npart=npart,
            gather=_make_gather(p["gn"], p["tail"], start),
            scatter=_make_scatter(p["sn"], p["tail"], start),
            edge=_tc_edge(npart, start // _EDGE_BLK),
        ))
        start += npart
    return parts


_PART_KERNELS = _build_parts()


def kernel(x, edge_index, edge_attr, calibrated_vlm_conf, W1, b1, W2, b2, Wa, ba):
    x = x.astype(jnp.float32)
    ei = edge_index.astype(jnp.int32)
    row = ei[0]
    col = ei[1]
    w1s = W1[:CH]
    w1t = W1[CH:2 * CH]
    w1e = W1[2 * CH:]
    wa1 = Wa[:CH]
    scal = jnp.stack([Wa[CH, 0], ba[0]]).reshape(1, 2)
    b1r = b1.reshape(1, CH)
    b2r = b2.reshape(1, CH)

    xa, xb = _tc_node(x, w1s, w1t)
    partials = []
    for part in _PART_KERNELS:
        g = part["gather"](xa, xb, row, col)
        gated = part["edge"](edge_attr, g, calibrated_vlm_conf, w1e,
                             b1r, W2, b2r, wa1, scal)
        partials.append(part["scatter"](gated, col))
    return _tc_add(partials)


# speedup vs baseline: 1.1323x; 1.1323x over previous
import functools

import jax
import jax.numpy as jnp
from jax import lax
from jax.experimental import pallas as pl
from jax.experimental.pallas import tpu as pltpu
from jax.experimental.pallas import tpu_sc as plsc

N_NODES = 10000
N_EDGES = 320000
CH = 128

NC = 2
NS = 16
NL = 16
NW = NC * NS
EPW = N_EDGES // NW
CHUNK = 128
ROWS_PER_TILE = 624
TAIL_ROWS = N_NODES - NS * ROWS_PER_TILE

_PARTS = (
    dict(gn=21, sn=21, tail=0),
    dict(gn=21, sn=21, tail=0),
    dict(gn=21, sn=21, tail=0),
    dict(gn=15, sn=15, tail=16),
)
_EDGE_BLK = 2048

_sc_mesh = plsc.VectorSubcoreMesh(core_axis_name="c", subcore_axis_name="s")


GC = 128
_RING = 3
_LOOK = 2


def _make_gather(nfull, tail, start):
    epw = nfull * GC + tail
    npart = NW * epw

    @functools.partial(
        pl.kernel,
        out_type=jax.ShapeDtypeStruct((npart, CH), jnp.float32),
        mesh=_sc_mesh,
        scratch_types=[
            pltpu.VMEM((_RING, GC), jnp.int32),
            pltpu.VMEM((_RING, GC), jnp.int32),
            pltpu.VMEM((_RING, GC, CH), jnp.float32),
            pltpu.VMEM((_RING, GC, CH), jnp.float32),
        ] + [pltpu.SemaphoreType.DMA] * (2 * _RING),
    )
    def gather_part(xa_hbm, xb_hbm, row_hbm, col_hbm, out_hbm,
                    idx_a, idx_b, buf_a, buf_b, *sems):
        sems_g = sems[:_RING]
        sems_w = sems[_RING:]
        wid = lax.axis_index("s") * NC + lax.axis_index("c")
        gbase = start + wid * epw
        obase = wid * epw

        def load_idx(j, b):
            off = gbase + j * GC
            pltpu.sync_copy(row_hbm.at[pl.ds(off, GC)], idx_a.at[b])
            pltpu.sync_copy(col_hbm.at[pl.ds(off, GC)], idx_b.at[b])

        def issue_gather(b):
            pltpu.async_copy(xa_hbm.at[idx_a.at[b]], buf_a.at[b], sems_g[b])
            pltpu.async_copy(xb_hbm.at[idx_b.at[b]], buf_b.at[b], sems_g[b])

        def wait_gather(b):
            pltpu.make_async_copy(xa_hbm.at[idx_a.at[b]], buf_a.at[b],
                                  sems_g[b]).wait()
            pltpu.make_async_copy(xb_hbm.at[idx_b.at[b]], buf_b.at[b],
                                  sems_g[b]).wait()

        def issue_write(j, b):
            off = obase + j * GC
            pltpu.async_copy(buf_a.at[b], out_hbm.at[pl.ds(off, GC)], sems_w[b])

        def wait_write(j, b):
            off = obase + j * GC
            pltpu.make_async_copy(buf_a.at[b], out_hbm.at[pl.ds(off, GC)],
                                  sems_w[b]).wait()

        def add_rows(b, n):
            def add_row(r, c2):
                for g in range(CH // NL):
                    sl = pl.ds(g * NL, NL)
                    buf_a[b, r, sl] = buf_a[b, r, sl] + buf_b[b, r, sl]
                return c2
            lax.fori_loop(0, n, add_row, 0, unroll=4)

        for b in range(_LOOK):
            load_idx(b, b)
            issue_gather(b)

        def group_body(g, carry):
            for b in range(_RING):
                j = g * _RING + b
                bn = (b + _LOOK) % _RING

                @pl.when(j + _LOOK < nfull)
                def _refill():
                    @pl.when(j >= _RING - _LOOK)
                    def _drain_prev_write():
                        wait_write(j - (_RING - _LOOK), bn)
                    load_idx(j + _LOOK, bn)
                    issue_gather(bn)

                wait_gather(b)
                add_rows(b, GC)
                issue_write(j, b)
            return carry

        lax.fori_loop(0, nfull // _RING, group_body, 0)
        for b in range(_RING):
            wait_write(nfull - _RING + b, b)

        if tail:
            toff_g = gbase + nfull * GC
            toff_o = obase + nfull * GC
            tsl = pl.ds(0, tail)
            pltpu.sync_copy(row_hbm.at[pl.ds(toff_g, tail)], idx_a.at[0, tsl])
            pltpu.sync_copy(col_hbm.at[pl.ds(toff_g, tail)], idx_b.at[0, tsl])
            pltpu.async_copy(xa_hbm.at[idx_a.at[0, tsl]], buf_a.at[0, tsl],
                             sems_g[0]).wait()
            pltpu.async_copy(xb_hbm.at[idx_b.at[0, tsl]], buf_b.at[0, tsl],
                             sems_g[1]).wait()
            add_rows(0, tail)
            pltpu.sync_copy(buf_a.at[0, tsl], out_hbm.at[pl.ds(toff_o, tail)])

    return gather_part


def _make_scatter(nfull, tail, start):
    epw = nfull * CHUNK + tail
    npart = NW * epw

    @functools.partial(
        pl.kernel,
        out_type=jax.ShapeDtypeStruct((NC, N_NODES, CH), jnp.float32),
        mesh=_sc_mesh,
        scratch_types=[
            pltpu.VMEM((3, CHUNK), jnp.int32),
            pltpu.VMEM((16,), jnp.int32),
            pltpu.VMEM((3, CHUNK, CH), jnp.float32),
            pltpu.VMEM_SHARED((N_NODES, CH), jnp.float32),
        ] + [pltpu.SemaphoreType.DMA] * 6,
    )
    def scatter_part(gated_hbm, col_hbm, out_hbm, idx_v, idx_t, buf, acc,
                     *sems):
        sems_l = sems[:3]
        sems_s = sems[3:]
        cid = lax.axis_index("c")
        sid = lax.axis_index("s")
        wid = sid * NC + cid

        def zrow(r, c2):
            for g in range(CH // NL):
                buf[0, r, pl.ds(g * NL, NL)] = jnp.zeros((NL,), jnp.float32)
            return c2

        lax.fori_loop(0, CHUNK, zrow, 0)
        for k in range(ROWS_PER_TILE // CHUNK):
            pltpu.sync_copy(buf.at[0],
                            acc.at[pl.ds(sid * ROWS_PER_TILE + k * CHUNK,
                                         CHUNK)])
        _zrem = ROWS_PER_TILE % CHUNK
        pltpu.sync_copy(buf.at[0, pl.ds(0, _zrem)],
                        acc.at[pl.ds(sid * ROWS_PER_TILE + ROWS_PER_TILE
                                     - _zrem, _zrem)])

        @pl.when(sid == 0)
        def _zero_tail():
            pltpu.sync_copy(buf.at[0, pl.ds(0, TAIL_ROWS)],
                            acc.at[pl.ds(NS * ROWS_PER_TILE, TAIL_ROWS)])

        plsc.subcore_barrier()

        gbase = start + wid * epw
        pbase = wid * epw

        def load_chunk(j, b):
            pltpu.sync_copy(col_hbm.at[pl.ds(gbase + j * CHUNK, CHUNK)],
                            idx_v.at[b])
            pltpu.async_copy(gated_hbm.at[pl.ds(pbase + j * CHUNK, CHUNK)],
                             buf.at[b], sems_l[b])

        def wait_load(j, b):
            pltpu.make_async_copy(gated_hbm.at[pl.ds(pbase + j * CHUNK, CHUNK)],
                                  buf.at[b], sems_l[b]).wait()

        def wait_scatter(b):
            pltpu.make_async_copy(buf.at[b], acc.at[idx_v.at[b]],
                                  sems_s[b]).wait()

        load_chunk(0, 0)
        load_chunk(1, 1)

        def group_body(g, carry):
            for b in range(3):
                j = g * 3 + b
                bn = (b + 2) % 3

                @pl.when(j + 2 < nfull)
                def _refill():
                    @pl.when(j >= 1)
                    def _drain_prev():
                        wait_scatter(bn)
                    load_chunk(j + 2, bn)

                wait_load(j, b)
                pltpu.async_copy(buf.at[b], acc.at[idx_v.at[b]], sems_s[b],
                                 add=True)
            return carry

        lax.fori_loop(0, nfull // 3, group_body, 0)
        for b in range(3):
            wait_scatter(b)

        if tail:
            tsl = pl.ds(0, tail)
            pltpu.sync_copy(col_hbm.at[pl.ds(gbase + nfull * CHUNK, tail)],
                            idx_t)
            pltpu.sync_copy(gated_hbm.at[pl.ds(pbase + nfull * CHUNK, tail)],
                            buf.at[0, tsl])
            pltpu.sync_copy(buf.at[0, tsl], acc.at[idx_t], add=True)

        plsc.subcore_barrier()
        pltpu.sync_copy(acc.at[pl.ds(sid * ROWS_PER_TILE, ROWS_PER_TILE)],
                        out_hbm.at[cid, pl.ds(sid * ROWS_PER_TILE,
                                              ROWS_PER_TILE)])

        @pl.when(sid == 0)
        def _copy_tail():
            pltpu.sync_copy(acc.at[pl.ds(NS * ROWS_PER_TILE, TAIL_ROWS)],
                            out_hbm.at[cid, pl.ds(NS * ROWS_PER_TILE,
                                                  TAIL_ROWS)])

    return scatter_part


_NODE_BLK = 2000


def _tc_node_body(x_ref, w1s_ref, w1t_ref, xa_ref, xb_ref):
    xv = x_ref[...]
    xa_ref[...] = jnp.dot(xv, w1s_ref[...], preferred_element_type=jnp.float32)
    xb_ref[...] = jnp.dot(xv, w1t_ref[...], preferred_element_type=jnp.float32)


def _tc_node(x, w1s, w1t):
    n_blk = N_NODES // _NODE_BLK
    return pl.pallas_call(
        _tc_node_body,
        grid=(n_blk,),
        in_specs=[
            pl.BlockSpec((_NODE_BLK, CH), lambda i: (i, 0)),
            pl.BlockSpec((CH, CH), lambda i: (0, 0)),
            pl.BlockSpec((CH, CH), lambda i: (0, 0)),
        ],
        out_specs=[
            pl.BlockSpec((_NODE_BLK, CH), lambda i: (i, 0)),
            pl.BlockSpec((_NODE_BLK, CH), lambda i: (i, 0)),
        ],
        out_shape=[
            jax.ShapeDtypeStruct((N_NODES, CH), jnp.float32),
            jax.ShapeDtypeStruct((N_NODES, CH), jnp.float32),
        ],
        compiler_params=pltpu.CompilerParams(
            dimension_semantics=("parallel",)),
    )(x, w1s, w1t)


def _tc_edge_body(ea_ref, g_ref, conf_ref, w1e_ref, b1_ref, w2_ref,
                  b2_ref, wa_ref, s_ref, out_ref):
    pre = (jnp.dot(ea_ref[...], w1e_ref[...], preferred_element_type=jnp.float32)
           + g_ref[...] + b1_ref[...])
    h = jnp.maximum(pre, 0.0)
    msg = jnp.dot(h, w2_ref[...], preferred_element_type=jnp.float32) + b2_ref[...]
    logit = (jnp.dot(msg, wa_ref[...], preferred_element_type=jnp.float32)
             + conf_ref[...] * s_ref[0, 0] + s_ref[0, 1])
    out_ref[...] = msg * jax.nn.sigmoid(logit)


def _tc_edge(npart, start_blk):
    n_blk = (npart + _EDGE_BLK - 1) // _EDGE_BLK

    def call(edge_attr, g, conf, w1e, b1, w2, b2, wa1, scal):
        return pl.pallas_call(
            _tc_edge_body,
            grid=(n_blk,),
            in_specs=[
                pl.BlockSpec((_EDGE_BLK, CH), lambda i: (i + start_blk, 0)),
                pl.BlockSpec((_EDGE_BLK, CH), lambda i: (i, 0)),
                pl.BlockSpec((_EDGE_BLK, 1), lambda i: (i + start_blk, 0)),
                pl.BlockSpec((CH, CH), lambda i: (0, 0)),
                pl.BlockSpec((1, CH), lambda i: (0, 0)),
                pl.BlockSpec((CH, CH), lambda i: (0, 0)),
                pl.BlockSpec((1, CH), lambda i: (0, 0)),
                pl.BlockSpec((CH, 1), lambda i: (0, 0)),
                pl.BlockSpec(memory_space=pltpu.SMEM),
            ],
            out_specs=pl.BlockSpec((_EDGE_BLK, CH), lambda i: (i, 0)),
            out_shape=jax.ShapeDtypeStruct((npart, CH), jnp.float32),
            compiler_params=pltpu.CompilerParams(
                dimension_semantics=("parallel",)),
        )(edge_attr, g, conf, w1e, b1, w2, b2, wa1, scal)

    return call


def _tc_add_body(p0_ref, p1_ref, p2_ref, p3_ref, out_ref):
    out_ref[...] = ((p0_ref[0] + p0_ref[1]) + (p1_ref[0] + p1_ref[1])
                    + ((p2_ref[0] + p2_ref[1]) + (p3_ref[0] + p3_ref[1])))


def _tc_add(partials):
    n_blk = N_NODES // _NODE_BLK
    spec = pl.BlockSpec((NC, _NODE_BLK, CH), lambda i: (0, i, 0))
    return pl.pallas_call(
        _tc_add_body,
        grid=(n_blk,),
        in_specs=[spec, spec, spec, spec],
        out_specs=pl.BlockSpec((_NODE_BLK, CH), lambda i: (i, 0)),
        out_shape=jax.ShapeDtypeStruct((N_NODES, CH), jnp.float32),
        compiler_params=pltpu.CompilerParams(
            dimension_semantics=("parallel",)),
    )(*partials)


def _build_parts():
    parts = []
    start = 0
    for p in _PARTS:
        epw = p["sn"] * CHUNK + p["tail"]
        npart = NW * epw
        parts.append(dict(
            start=start,
            npart=npart,
            gather=_make_gather(p["gn"], p["tail"], start),
            scatter=_make_scatter(p["sn"], p["tail"], start),
            edge=_tc_edge(npart, start // _EDGE_BLK),
        ))
        start += npart
    return parts


_PART_KERNELS = _build_parts()


def kernel(x, edge_index, edge_attr, calibrated_vlm_conf, W1, b1, W2, b2, Wa, ba):
    x = x.astype(jnp.float32)
    ei = edge_index.astype(jnp.int32)
    row = ei[0]
    col = ei[1]
    w1s = W1[:CH]
    w1t = W1[CH:2 * CH]
    w1e = W1[2 * CH:]
    wa1 = Wa[:CH]
    scal = jnp.stack([Wa[CH, 0], ba[0]]).reshape(1, 2)
    b1r = b1.reshape(1, CH)
    b2r = b2.reshape(1, CH)

    xa, xb = _tc_node(x, w1s, w1t)
    partials = []
    for part in _PART_KERNELS:
        g = part["gather"](xa, xb, row, col)
        gated = part["edge"](edge_attr, g, calibrated_vlm_conf, w1e,
                             b1r, W2, b2r, wa1, scal)
        partials.append(part["scatter"](gated, col))
    return _tc_add(partials)
